# baseline (device time: 28833 ns/iter reference)
import jax
import jax.numpy as jnp
from jax import lax
from jax.experimental import pallas as pl
from jax.experimental.pallas import tpu as pltpu

N_Y = 4


def kernel(partial, gamma):
    _, m_tot, d = partial.shape
    m_per = m_tot // N_Y

    def body(p_ref, g_ref, out_ref, comm_ref, send_sems, recv_sems):
        my_x = lax.axis_index("x")
        my_y = lax.axis_index("y")
        my_z = lax.axis_index("z")
        left = (my_y - 1) % N_Y
        right = (my_y + 1) % N_Y

        barrier_sem = pltpu.get_barrier_semaphore()
        for nbr in [left, right]:
            pl.semaphore_signal(
                barrier_sem, inc=1,
                device_id=(my_x, nbr, my_z),
                device_id_type=pl.DeviceIdType.MESH,
            )
        pl.semaphore_wait(barrier_sem, 2)

        def local_chunk(c):
            return p_ref[0, pl.ds(c * m_per, m_per), :]

        comm_ref[0] = local_chunk((my_y - 1) % N_Y).astype(jnp.bfloat16)

        for s in range(N_Y - 1):
            rdma = pltpu.make_async_remote_copy(
                src_ref=comm_ref.at[s],
                dst_ref=comm_ref.at[s + 1],
                send_sem=send_sems.at[s],
                recv_sem=recv_sems.at[s],
                device_id=(my_x, right, my_z),
                device_id_type=pl.DeviceIdType.MESH,
            )
            rdma.start()
            rdma.wait()
            c_recv = (my_y - s - 2) % N_Y
            if s < N_Y - 2:
                comm_ref[s + 1] = (
                    comm_ref[s + 1] + local_chunk(c_recv).astype(jnp.bfloat16)
                )

        y = comm_ref[N_Y - 1].astype(jnp.float32) + local_chunk(my_y)
        ms = jnp.mean(y * y, axis=-1, keepdims=True) + 1e-6
        out_ref[...] = y * lax.rsqrt(ms) * g_ref[...]

    return pl.pallas_call(
        body,
        out_shape=jax.ShapeDtypeStruct((m_per, d), jnp.float32),
        in_specs=[
            pl.BlockSpec(memory_space=pltpu.VMEM),
            pl.BlockSpec(memory_space=pltpu.VMEM),
        ],
        out_specs=pl.BlockSpec(memory_space=pltpu.VMEM),
        scratch_shapes=[
            pltpu.VMEM((N_Y, m_per, d), jnp.bfloat16),
            pltpu.SemaphoreType.DMA((N_Y - 1,)),
            pltpu.SemaphoreType.DMA((N_Y - 1,)),
        ],
        compiler_params=pltpu.CompilerParams(collective_id=0),
    )(partial, gamma.reshape(1, d))


# device time: 24691 ns/iter; 1.1678x vs baseline; 1.1678x over previous
import functools

import jax
import jax.numpy as jnp
from jax import lax
from jax.experimental import pallas as pl
from jax.experimental.pallas import tpu as pltpu

N_X = 2
N_Y = 4
N_Z = 4
N_REP = N_X * N_Z

_XZ_OFFSETS = [(ox, oz) for ox in range(N_X) for oz in range(N_Z) if (ox, oz) != (0, 0)]


def kernel(partial, gamma):
    _, m_tot, d = partial.shape
    m_per = m_tot // N_Y
    m_sub = m_per // N_REP

    def body(p_ref, g_ref, out_ref, sbuf, rbuf, gbuf, ssem1, rsem1, ssem2, rsem2):
        my_x = lax.axis_index("x")
        my_y = lax.axis_index("y")
        my_z = lax.axis_index("z")
        my_r = N_Z * my_x + my_z

        barrier_sem = pltpu.get_barrier_semaphore()
        for o in range(1, N_Y):
            pl.semaphore_signal(
                barrier_sem, inc=1,
                device_id=(my_x, (my_y + o) % N_Y, my_z),
                device_id_type=pl.DeviceIdType.MESH,
            )
        for ox, oz in _XZ_OFFSETS:
            pl.semaphore_signal(
                barrier_sem, inc=1,
                device_id=((my_x + ox) % N_X, my_y, (my_z + oz) % N_Z),
                device_id_type=pl.DeviceIdType.MESH,
            )
        pl.semaphore_wait(barrier_sem, N_Y - 1 + N_REP - 1)

        p1 = []
        for o in range(1, N_Y):
            ty = (my_y + o) % N_Y
            sbuf[o - 1] = p_ref[
                0, pl.ds((m_per * ty + m_sub * my_r), m_sub), :
            ].astype(jnp.bfloat16)
            rdma = pltpu.make_async_remote_copy(
                src_ref=sbuf.at[o - 1],
                dst_ref=rbuf.at[o - 1],
                send_sem=ssem1.at[o - 1],
                recv_sem=rsem1.at[o - 1],
                device_id=(my_x, ty, my_z),
                device_id_type=pl.DeviceIdType.MESH,
            )
            rdma.start()
            p1.append(rdma)

        y32 = p_ref[0, pl.ds(m_per * my_y + m_sub * my_r, m_sub), :]
        for o in range(1, N_Y):
            p1[o - 1].wait_recv()
            y32 = y32 + rbuf[o - 1].astype(jnp.float32)

        ms = jnp.mean(y32 * y32, axis=-1, keepdims=True) + 1e-6
        normed = y32 * lax.rsqrt(ms) * g_ref[...]
        gbuf[my_r] = normed.astype(jnp.bfloat16)

        p2_send, p2_recv = [], []
        for ox, oz in _XZ_OFFSETS:
            tx = (my_x + ox) % N_X
            tz = (my_z + oz) % N_Z
            s = N_Z * tx + tz
            send = pltpu.make_async_remote_copy(
                src_ref=gbuf.at[my_r],
                dst_ref=gbuf.at[my_r],
                send_sem=ssem2.at[s],
                recv_sem=rsem2.at[my_r],
                device_id=(tx, my_y, tz),
                device_id_type=pl.DeviceIdType.MESH,
            )
            send.start()
            p2_send.append(send)
            recv = pltpu.make_async_remote_copy(
                src_ref=gbuf.at[s],
                dst_ref=gbuf.at[s],
                send_sem=ssem2.at[s],
                recv_sem=rsem2.at[s],
                device_id=(tx, my_y, tz),
                device_id_type=pl.DeviceIdType.MESH,
            )
            p2_recv.append((recv, s))

        out_ref[pl.ds(m_sub * my_r, m_sub), :] = normed

        for recv, s in p2_recv:
            recv.wait_recv()
            out_ref[pl.ds(m_sub * s, m_sub), :] = gbuf[s].astype(jnp.float32)

        for rdma in p1:
            rdma.wait_send()
        for rdma in p2_send:
            rdma.wait_send()

        @functools.partial(pl.run_scoped, exit_sem=pltpu.SemaphoreType.REGULAR)
        def _(exit_sem):
            for o in range(1, N_Y):
                pl.semaphore_signal(
                    exit_sem, inc=1,
                    device_id=(my_x, (my_y + o) % N_Y, my_z),
                    device_id_type=pl.DeviceIdType.MESH,
                )
            for ox, oz in _XZ_OFFSETS:
                pl.semaphore_signal(
                    exit_sem, inc=1,
                    device_id=((my_x + ox) % N_X, my_y, (my_z + oz) % N_Z),
                    device_id_type=pl.DeviceIdType.MESH,
                )
            pl.semaphore_wait(exit_sem, N_Y - 1 + N_REP - 1)

    return pl.pallas_call(
        body,
        out_shape=jax.ShapeDtypeStruct((m_per, d), jnp.float32),
        in_specs=[
            pl.BlockSpec(memory_space=pltpu.VMEM),
            pl.BlockSpec(memory_space=pltpu.VMEM),
        ],
        out_specs=pl.BlockSpec(memory_space=pltpu.VMEM),
        scratch_shapes=[
            pltpu.VMEM((N_Y - 1, m_sub, d), jnp.bfloat16),
            pltpu.VMEM((N_Y - 1, m_sub, d), jnp.bfloat16),
            pltpu.VMEM((N_REP, m_sub, d), jnp.bfloat16),
            pltpu.SemaphoreType.DMA((N_Y - 1,)),
            pltpu.SemaphoreType.DMA((N_Y - 1,)),
            pltpu.SemaphoreType.DMA((N_REP,)),
            pltpu.SemaphoreType.DMA((N_REP,)),
        ],
        compiler_params=pltpu.CompilerParams(collective_id=0),
    )(partial, gamma.reshape(1, d))


# device time: 21179 ns/iter; 1.3614x vs baseline; 1.1658x over previous
import jax
import jax.numpy as jnp
from jax import lax
from jax.experimental import pallas as pl
from jax.experimental.pallas import tpu as pltpu

N_X = 2
N_Y = 4
N_Z = 4
N_REP = N_X * N_Z

_XZ_OFFSETS = [(0, 1), (0, 3), (1, 0), (0, 2), (1, 1), (1, 3), (1, 2)]


def kernel(partial, gamma):
    _, m_tot, d = partial.shape
    m_per = m_tot // N_Y
    m_sub = m_per // N_REP

    def body(p_ref, g_ref, out_ref, lbuf, sbuf, rbuf, gbuf,
             dma_sems, ssem1, rsem1, ssem2, rsem2, xz_sem):
        my_x = lax.axis_index("x")
        my_y = lax.axis_index("y")
        my_z = lax.axis_index("z")
        my_r = N_Z * my_x + my_z

        dmas = []
        for o in range(N_Y):
            ty = (my_y + o) % N_Y
            cp = pltpu.make_async_copy(
                p_ref.at[0, pl.ds(m_per * ty + m_sub * my_r, m_sub), :],
                lbuf.at[o],
                dma_sems.at[o],
            )
            cp.start()
            dmas.append(cp)

        barrier_sem = pltpu.get_barrier_semaphore()
        for o in range(1, N_Y):
            pl.semaphore_signal(
                barrier_sem, inc=1,
                device_id=(my_x, (my_y + o) % N_Y, my_z),
                device_id_type=pl.DeviceIdType.MESH,
            )
        for ox, oz in _XZ_OFFSETS:
            pl.semaphore_signal(
                xz_sem, inc=1,
                device_id=((my_x + ox) % N_X, my_y, (my_z + oz) % N_Z),
                device_id_type=pl.DeviceIdType.MESH,
            )

        for o in range(1, N_Y):
            dmas[o].wait()
            sbuf[o - 1] = lbuf[o].astype(jnp.bfloat16)

        pl.semaphore_wait(barrier_sem, N_Y - 1)

        p1 = []
        for o in range(1, N_Y):
            rdma = pltpu.make_async_remote_copy(
                src_ref=sbuf.at[o - 1],
                dst_ref=rbuf.at[o - 1],
                send_sem=ssem1.at[o - 1],
                recv_sem=rsem1.at[o - 1],
                device_id=(my_x, (my_y + o) % N_Y, my_z),
                device_id_type=pl.DeviceIdType.MESH,
            )
            rdma.start()
            p1.append(rdma)

        dmas[0].wait()
        y32 = lbuf[0, :, :]
        for o in range(1, N_Y):
            p1[o - 1].wait_recv()
            y32 = y32 + rbuf[o - 1].astype(jnp.float32)

        ms = jnp.mean(y32 * y32, axis=-1, keepdims=True) + 1e-6
        normed = y32 * lax.rsqrt(ms) * g_ref[...]
        gbuf[my_r] = normed.astype(jnp.bfloat16)

        pl.semaphore_wait(xz_sem, N_REP - 1)
        p2_send, p2_recv = [], []
        for ox, oz in _XZ_OFFSETS:
            tx = (my_x + ox) % N_X
            tz = (my_z + oz) % N_Z
            s = N_Z * tx + tz
            send = pltpu.make_async_remote_copy(
                src_ref=gbuf.at[my_r],
                dst_ref=gbuf.at[my_r],
                send_sem=ssem2.at[s],
                recv_sem=rsem2.at[my_r],
                device_id=(tx, my_y, tz),
                device_id_type=pl.DeviceIdType.MESH,
            )
            send.start()
            p2_send.append(send)
            recv = pltpu.make_async_remote_copy(
                src_ref=gbuf.at[s],
                dst_ref=gbuf.at[s],
                send_sem=ssem2.at[s],
                recv_sem=rsem2.at[s],
                device_id=(tx, my_y, tz),
                device_id_type=pl.DeviceIdType.MESH,
            )
            p2_recv.append((recv, s))

        out_ref[pl.ds(m_sub * my_r, m_sub), :] = normed

        for recv, s in p2_recv:
            recv.wait_recv()
            out_ref[pl.ds(m_sub * s, m_sub), :] = gbuf[s].astype(jnp.float32)

        for rdma in p1:
            rdma.wait_send()
        for rdma in p2_send:
            rdma.wait_send()

    return pl.pallas_call(
        body,
        out_shape=jax.ShapeDtypeStruct((m_per, d), jnp.float32),
        in_specs=[
            pl.BlockSpec(memory_space=pltpu.MemorySpace.HBM),
            pl.BlockSpec(memory_space=pltpu.VMEM),
        ],
        out_specs=pl.BlockSpec(memory_space=pltpu.VMEM),
        scratch_shapes=[
            pltpu.VMEM((N_Y, m_sub, d), jnp.float32),
            pltpu.VMEM((N_Y - 1, m_sub, d), jnp.bfloat16),
            pltpu.VMEM((N_Y - 1, m_sub, d), jnp.bfloat16),
            pltpu.VMEM((N_REP, m_sub, d), jnp.bfloat16),
            pltpu.SemaphoreType.DMA((N_Y,)),
            pltpu.SemaphoreType.DMA((N_Y - 1,)),
            pltpu.SemaphoreType.DMA((N_Y - 1,)),
            pltpu.SemaphoreType.DMA((N_REP,)),
            pltpu.SemaphoreType.DMA((N_REP,)),
            pltpu.SemaphoreType.REGULAR,
        ],
        compiler_params=pltpu.CompilerParams(collective_id=0),
    )(partial, gamma.reshape(1, d))
